# baseline (device time: 17971 ns/iter reference)
import jax
import jax.numpy as jnp
from jax import lax
from jax.experimental import pallas as pl
from jax.experimental.pallas import tpu as pltpu

T = 1024
SND_ROWS = 576
CH = 32
M32 = SND_ROWS // CH
M8 = 3


def _align8(v):
    return (v // 8) * 8


def _exchange(x, dest2d, c2d, c0_arr):
    d = x.shape[1]

    def body(c0_ref, dest_ref, c_ref, x_ref, out_ref,
             snd_scr, rcv_scr, send32, recv32, send8, recv8):
        my_x = lax.axis_index("x")
        my_y = lax.axis_index("y")
        my_z = lax.axis_index("z")
        partner = (my_x, 1 - my_y, my_z)
        y0 = my_y == 0

        c0v = c0_ref[0]
        n_send = jnp.where(y0, T - c0v, c0v)
        recv_start = jnp.where(y0, c0v, 0)
        r_dst = jnp.where(y0, 0, (T - c0v) % 8)
        l8_out = _align8(r_dst + n_send + 7)
        m32_out = l8_out // CH
        rem8_out = (l8_out % CH) // 8
        r_fix = recv_start % 8
        l8_in = _align8(r_fix + n_send + 7)
        m32_in = l8_in // CH
        rem8_in = (l8_in % CH) // 8

        barrier = pltpu.get_barrier_semaphore()
        pl.semaphore_signal(
            barrier, inc=1, device_id=partner,
            device_id_type=pl.DeviceIdType.MESH,
        )

        c = c_ref[:, :]
        ii = lax.broadcasted_iota(jnp.int32, (1, T), 1)
        cum1 = ii + 1 - c
        keep = dest_ref[:, :] == my_y
        fwd_k = jnp.where(keep, jnp.where(y0, c - 1, c0v + cum1 - 1), -1)
        fwd_s = jnp.where(keep, -1,
                          jnp.where(y0, cum1 - 1, r_dst + c - 1))

        xb = x_ref[:, :].astype(jnp.bfloat16)

        oh_s = (lax.broadcasted_iota(jnp.int32, (SND_ROWS, T), 0)
                == fwd_s).astype(jnp.bfloat16)
        snd_scr[:, :] = jax.lax.dot_general(
            oh_s, xb, (((1,), (0,)), ((), ())),
            preferred_element_type=jnp.float32).astype(jnp.bfloat16)

        pl.semaphore_wait(barrier, 1)

        for k in range(M32):
            @pl.when(k < m32_out)
            def _(k=k):
                pltpu.make_async_remote_copy(
                    src_ref=snd_scr.at[pl.ds(k * CH, CH), :],
                    dst_ref=rcv_scr.at[pl.ds(k * CH, CH), :],
                    send_sem=send32.at[k],
                    recv_sem=recv32.at[k],
                    device_id=partner,
                    device_id_type=pl.DeviceIdType.MESH,
                ).start()
        for j in range(M8):
            @pl.when(j < rem8_out)
            def _(j=j):
                off = pl.multiple_of(m32_out * CH + j * 8, 8)
                pltpu.make_async_remote_copy(
                    src_ref=snd_scr.at[pl.ds(off, 8), :],
                    dst_ref=rcv_scr.at[pl.ds(off, 8), :],
                    send_sem=send8.at[j],
                    recv_sem=recv8.at[j],
                    device_id=partner,
                    device_id_type=pl.DeviceIdType.MESH,
                ).start()

        oh_k = (lax.broadcasted_iota(jnp.int32, (T, T), 0)
                == fwd_k).astype(jnp.bfloat16)
        out_ref[:, :] = jax.lax.dot_general(
            oh_k, xb, (((1,), (0,)), ((), ())),
            preferred_element_type=jnp.float32)

        def desc32(k):
            return pltpu.make_async_remote_copy(
                src_ref=snd_scr.at[pl.ds(0, CH), :],
                dst_ref=rcv_scr.at[pl.ds(0, CH), :],
                send_sem=send32.at[k], recv_sem=recv32.at[k],
                device_id=partner, device_id_type=pl.DeviceIdType.MESH,
            )

        def desc8(j):
            return pltpu.make_async_remote_copy(
                src_ref=snd_scr.at[pl.ds(0, 8), :],
                dst_ref=rcv_scr.at[pl.ds(0, 8), :],
                send_sem=send8.at[j], recv_sem=recv8.at[j],
                device_id=partner, device_id_type=pl.DeviceIdType.MESH,
            )

        for k in range(M32):
            @pl.when(k < m32_in)
            def _(k=k):
                desc32(k).wait_recv()
        for j in range(M8):
            @pl.when(j < rem8_in)
            def _(j=j):
                desc8(j).wait_recv()

        a_h = pl.multiple_of(_align8(recv_start), 8)

        def blend(off, rows, it):
            q = it + off
            mask = (q >= r_fix) & (q < r_fix + n_send)
            dst = pl.ds(pl.multiple_of(a_h + off, 8), rows)
            out_ref[dst, :] = jnp.where(
                mask,
                rcv_scr[pl.ds(off, rows), :].astype(jnp.float32),
                out_ref[dst, :])

        it32 = lax.broadcasted_iota(jnp.int32, (CH, d), 0)
        it8 = lax.broadcasted_iota(jnp.int32, (8, d), 0)
        for k in range(M32):
            @pl.when(k < m32_in)
            def _(k=k):
                blend(k * CH, CH, it32)
        for j in range(M8):
            @pl.when(j < rem8_in)
            def _(j=j):
                blend(m32_in * CH + j * 8, 8, it8)

        for k in range(M32):
            @pl.when(k < m32_out)
            def _(k=k):
                desc32(k).wait_send()
        for j in range(M8):
            @pl.when(j < rem8_out)
            def _(j=j):
                desc8(j).wait_send()

    return pl.pallas_call(
        body,
        out_shape=jax.ShapeDtypeStruct((T, d), x.dtype),
        in_specs=[
            pl.BlockSpec(memory_space=pltpu.SMEM),
            pl.BlockSpec(memory_space=pltpu.VMEM),
            pl.BlockSpec(memory_space=pltpu.VMEM),
            pl.BlockSpec(memory_space=pltpu.VMEM),
        ],
        out_specs=pl.BlockSpec(memory_space=pltpu.VMEM),
        scratch_shapes=[
            pltpu.VMEM((SND_ROWS, d), jnp.bfloat16),
            pltpu.VMEM((SND_ROWS, d), jnp.bfloat16),
            pltpu.SemaphoreType.DMA((M32,)),
            pltpu.SemaphoreType.DMA((M32,)),
            pltpu.SemaphoreType.DMA((M8,)),
            pltpu.SemaphoreType.DMA((M8,)),
        ],
        compiler_params=pltpu.CompilerParams(collective_id=0),
    )(c0_arr, dest2d, c2d, x)


def kernel(x, dest):
    t, d = x.shape
    c = jnp.cumsum((dest == 0).astype(jnp.int32))
    c0 = c[-1]
    return _exchange(x, dest.reshape(1, t), c.reshape(1, t),
                     jnp.reshape(c0, (1,)))


# device time: 16797 ns/iter; 1.0699x vs baseline; 1.0699x over previous
import jax
import jax.numpy as jnp
from jax import lax
from jax.experimental import pallas as pl
from jax.experimental.pallas import tpu as pltpu

T = 1024
SND_ROWS = 576
HALF = 288
CH = 32
M32 = SND_ROWS // CH
M8 = 3


def _align8(v):
    return (v // 8) * 8


def _exchange(xb, k_idx, s_idx, c0_arr, out_dtype):
    d = xb.shape[1]

    def body(c0_ref, kidx_ref, sidx_ref, x_ref, out_ref,
             snd_scr, rcv_scr, send32, recv32, send8, recv8):
        my_x = lax.axis_index("x")
        my_y = lax.axis_index("y")
        my_z = lax.axis_index("z")
        partner = (my_x, 1 - my_y, my_z)
        y0 = my_y == 0

        c0v = c0_ref[0]
        n_send = jnp.where(y0, T - c0v, c0v)
        recv_start = jnp.where(y0, c0v, 0)
        r_dst = jnp.where(y0, 0, (T - c0v) % 8)
        l8_out = _align8(r_dst + n_send + 7)
        m32_out = l8_out // CH
        rem8_out = (l8_out % CH) // 8
        r_fix = recv_start % 8
        l8_in = _align8(r_fix + n_send + 7)
        m32_in = l8_in // CH
        rem8_in = (l8_in % CH) // 8

        barrier = pltpu.get_barrier_semaphore()
        pl.semaphore_signal(
            barrier, inc=1, device_id=partner,
            device_id_type=pl.DeviceIdType.MESH,
        )

        xv = x_ref[:, :]

        oh_s0 = (lax.broadcasted_iota(jnp.int32, (HALF, T), 0)
                 == sidx_ref[:, :]).astype(jnp.bfloat16)
        snd_scr[pl.ds(0, HALF), :] = jax.lax.dot_general(
            oh_s0, xv, (((1,), (0,)), ((), ())),
            preferred_element_type=jnp.float32).astype(jnp.bfloat16)

        pl.semaphore_wait(barrier, 1)

        def start32(k):
            pltpu.make_async_remote_copy(
                src_ref=snd_scr.at[pl.ds(k * CH, CH), :],
                dst_ref=rcv_scr.at[pl.ds(k * CH, CH), :],
                send_sem=send32.at[k],
                recv_sem=recv32.at[k],
                device_id=partner,
                device_id_type=pl.DeviceIdType.MESH,
            ).start()

        for k in range(HALF // CH):
            @pl.when(k < m32_out)
            def _(k=k):
                start32(k)

        oh_s1 = (lax.broadcasted_iota(jnp.int32, (SND_ROWS - HALF, T), 0)
                 == sidx_ref[:, :] - HALF).astype(jnp.bfloat16)
        snd_scr[pl.ds(HALF, SND_ROWS - HALF), :] = jax.lax.dot_general(
            oh_s1, xv, (((1,), (0,)), ((), ())),
            preferred_element_type=jnp.float32).astype(jnp.bfloat16)

        for k in range(HALF // CH, M32):
            @pl.when(k < m32_out)
            def _(k=k):
                start32(k)
        for j in range(M8):
            @pl.when(j < rem8_out)
            def _(j=j):
                off = pl.multiple_of(m32_out * CH + j * 8, 8)
                pltpu.make_async_remote_copy(
                    src_ref=snd_scr.at[pl.ds(off, 8), :],
                    dst_ref=rcv_scr.at[pl.ds(off, 8), :],
                    send_sem=send8.at[j],
                    recv_sem=recv8.at[j],
                    device_id=partner,
                    device_id_type=pl.DeviceIdType.MESH,
                ).start()

        oh_k = (lax.broadcasted_iota(jnp.int32, (T, T), 0)
                == kidx_ref[:, :]).astype(jnp.bfloat16)
        out_ref[:, :] = jax.lax.dot_general(
            oh_k, xv, (((1,), (0,)), ((), ())),
            preferred_element_type=jnp.float32)

        def desc32(k):
            return pltpu.make_async_remote_copy(
                src_ref=snd_scr.at[pl.ds(0, CH), :],
                dst_ref=rcv_scr.at[pl.ds(0, CH), :],
                send_sem=send32.at[k], recv_sem=recv32.at[k],
                device_id=partner, device_id_type=pl.DeviceIdType.MESH,
            )

        def desc8(j):
            return pltpu.make_async_remote_copy(
                src_ref=snd_scr.at[pl.ds(0, 8), :],
                dst_ref=rcv_scr.at[pl.ds(0, 8), :],
                send_sem=send8.at[j], recv_sem=recv8.at[j],
                device_id=partner, device_id_type=pl.DeviceIdType.MESH,
            )

        for k in range(M32):
            @pl.when(k < m32_in)
            def _(k=k):
                desc32(k).wait_recv()
        for j in range(M8):
            @pl.when(j < rem8_in)
            def _(j=j):
                desc8(j).wait_recv()

        a_h = pl.multiple_of(_align8(recv_start), 8)

        def blend(off, rows, it):
            q = it + off
            mask = (q >= r_fix) & (q < r_fix + n_send)
            dst = pl.ds(pl.multiple_of(a_h + off, 8), rows)
            out_ref[dst, :] = jnp.where(
                mask,
                rcv_scr[pl.ds(off, rows), :].astype(jnp.float32),
                out_ref[dst, :])

        it32 = lax.broadcasted_iota(jnp.int32, (CH, d), 0)
        it8 = lax.broadcasted_iota(jnp.int32, (8, d), 0)
        for k in range(M32):
            @pl.when(k < m32_in)
            def _(k=k):
                blend(k * CH, CH, it32)
        for j in range(M8):
            @pl.when(j < rem8_in)
            def _(j=j):
                blend(m32_in * CH + j * 8, 8, it8)

        for k in range(M32):
            @pl.when(k < m32_out)
            def _(k=k):
                desc32(k).wait_send()
        for j in range(M8):
            @pl.when(j < rem8_out)
            def _(j=j):
                desc8(j).wait_send()

    return pl.pallas_call(
        body,
        out_shape=jax.ShapeDtypeStruct((T, d), out_dtype),
        in_specs=[
            pl.BlockSpec(memory_space=pltpu.SMEM),
            pl.BlockSpec(memory_space=pltpu.VMEM),
            pl.BlockSpec(memory_space=pltpu.VMEM),
            pl.BlockSpec(memory_space=pltpu.VMEM),
        ],
        out_specs=pl.BlockSpec(memory_space=pltpu.VMEM),
        scratch_shapes=[
            pltpu.VMEM((SND_ROWS, d), jnp.bfloat16),
            pltpu.VMEM((SND_ROWS, d), jnp.bfloat16),
            pltpu.SemaphoreType.DMA((M32,)),
            pltpu.SemaphoreType.DMA((M32,)),
            pltpu.SemaphoreType.DMA((M8,)),
            pltpu.SemaphoreType.DMA((M8,)),
        ],
        compiler_params=pltpu.CompilerParams(collective_id=0),
    )(c0_arr, k_idx, s_idx, xb)


def kernel(x, dest):
    t, d = x.shape
    my_y = lax.axis_index("y")
    y0 = my_y == 0

    zeros = (dest == 0).astype(jnp.int32)
    c = jnp.cumsum(zeros)
    c0 = c[-1]
    i = jnp.arange(t, dtype=jnp.int32)
    cum1 = (i + 1) - c
    r_dst = jnp.where(y0, 0, (t - c0) % 8)

    keep_mask = jnp.where(y0, zeros == 1, zeros == 0)
    fwd_k = jnp.where(keep_mask,
                      jnp.where(y0, c - 1, c0 + cum1 - 1), -1)
    fwd_s = jnp.where(keep_mask, -1,
                      jnp.where(y0, cum1 - 1, r_dst + c - 1))

    return _exchange(x.astype(jnp.bfloat16), fwd_k.reshape(1, t),
                     fwd_s.reshape(1, t), jnp.reshape(c0, (1,)), x.dtype)


# device time: 16686 ns/iter; 1.0770x vs baseline; 1.0067x over previous
import jax
import jax.numpy as jnp
from jax import lax
from jax.experimental import pallas as pl
from jax.experimental.pallas import tpu as pltpu

T = 1024
SND_ROWS = 576
HALF = 288
CH = 32
M32 = SND_ROWS // CH
M8 = 3


def _align8(v):
    return (v // 8) * 8


def _exchange(xb, k_idx, s_idx, c0_arr, out_dtype):
    d = xb.shape[1]

    def body(c0_ref, kidx_ref, sidx_ref, x_ref, out_ref,
             snd_scr, rcv_scr, send32, recv32, send8, recv8):
        my_x = lax.axis_index("x")
        my_y = lax.axis_index("y")
        my_z = lax.axis_index("z")
        partner = (my_x, 1 - my_y, my_z)
        y0 = my_y == 0

        c0v = c0_ref[0]
        n_send = jnp.where(y0, T - c0v, c0v)
        recv_start = jnp.where(y0, c0v, 0)
        r_dst = jnp.where(y0, 0, (T - c0v) % 8)
        l8_out = _align8(r_dst + n_send + 7)
        m32_out = l8_out // CH
        rem8_out = (l8_out % CH) // 8
        r_fix = recv_start % 8
        l8_in = _align8(r_fix + n_send + 7)
        m32_in = l8_in // CH
        rem8_in = (l8_in % CH) // 8

        barrier = pltpu.get_barrier_semaphore()
        pl.semaphore_signal(
            barrier, inc=1, device_id=partner,
            device_id_type=pl.DeviceIdType.MESH,
        )

        xv = x_ref[:, :].astype(jnp.bfloat16)

        oh_s0 = (lax.broadcasted_iota(jnp.int32, (HALF, T), 0)
                 == sidx_ref[:, :]).astype(jnp.bfloat16)
        snd_scr[pl.ds(0, HALF), :] = jax.lax.dot_general(
            oh_s0, xv, (((1,), (0,)), ((), ())),
            preferred_element_type=jnp.float32).astype(jnp.bfloat16)

        pl.semaphore_wait(barrier, 1)

        def start32(k):
            pltpu.make_async_remote_copy(
                src_ref=snd_scr.at[pl.ds(k * CH, CH), :],
                dst_ref=rcv_scr.at[pl.ds(k * CH, CH), :],
                send_sem=send32.at[k],
                recv_sem=recv32.at[k],
                device_id=partner,
                device_id_type=pl.DeviceIdType.MESH,
            ).start()

        for k in range(HALF // CH):
            @pl.when(k < m32_out)
            def _(k=k):
                start32(k)

        oh_s1 = (lax.broadcasted_iota(jnp.int32, (SND_ROWS - HALF, T), 0)
                 == sidx_ref[:, :] - HALF).astype(jnp.bfloat16)
        snd_scr[pl.ds(HALF, SND_ROWS - HALF), :] = jax.lax.dot_general(
            oh_s1, xv, (((1,), (0,)), ((), ())),
            preferred_element_type=jnp.float32).astype(jnp.bfloat16)

        for k in range(HALF // CH, M32):
            @pl.when(k < m32_out)
            def _(k=k):
                start32(k)
        for j in range(M8):
            @pl.when(j < rem8_out)
            def _(j=j):
                off = pl.multiple_of(m32_out * CH + j * 8, 8)
                pltpu.make_async_remote_copy(
                    src_ref=snd_scr.at[pl.ds(off, 8), :],
                    dst_ref=rcv_scr.at[pl.ds(off, 8), :],
                    send_sem=send8.at[j],
                    recv_sem=recv8.at[j],
                    device_id=partner,
                    device_id_type=pl.DeviceIdType.MESH,
                ).start()

        oh_k = (lax.broadcasted_iota(jnp.int32, (T, T), 0)
                == kidx_ref[:, :]).astype(jnp.bfloat16)
        out_ref[:, :] = jax.lax.dot_general(
            oh_k, xv, (((1,), (0,)), ((), ())),
            preferred_element_type=jnp.float32)

        def desc32(k):
            return pltpu.make_async_remote_copy(
                src_ref=snd_scr.at[pl.ds(0, CH), :],
                dst_ref=rcv_scr.at[pl.ds(0, CH), :],
                send_sem=send32.at[k], recv_sem=recv32.at[k],
                device_id=partner, device_id_type=pl.DeviceIdType.MESH,
            )

        def desc8(j):
            return pltpu.make_async_remote_copy(
                src_ref=snd_scr.at[pl.ds(0, 8), :],
                dst_ref=rcv_scr.at[pl.ds(0, 8), :],
                send_sem=send8.at[j], recv_sem=recv8.at[j],
                device_id=partner, device_id_type=pl.DeviceIdType.MESH,
            )

        for k in range(M32):
            @pl.when(k < m32_in)
            def _(k=k):
                desc32(k).wait_recv()
        for j in range(M8):
            @pl.when(j < rem8_in)
            def _(j=j):
                desc8(j).wait_recv()

        a_h = pl.multiple_of(_align8(recv_start), 8)

        def blend(off, rows, it):
            q = it + off
            mask = (q >= r_fix) & (q < r_fix + n_send)
            dst = pl.ds(pl.multiple_of(a_h + off, 8), rows)
            out_ref[dst, :] = jnp.where(
                mask,
                rcv_scr[pl.ds(off, rows), :].astype(jnp.float32),
                out_ref[dst, :])

        it32 = lax.broadcasted_iota(jnp.int32, (CH, d), 0)
        it8 = lax.broadcasted_iota(jnp.int32, (8, d), 0)
        for k in range(M32):
            @pl.when(k < m32_in)
            def _(k=k):
                blend(k * CH, CH, it32)
        for j in range(M8):
            @pl.when(j < rem8_in)
            def _(j=j):
                blend(m32_in * CH + j * 8, 8, it8)

        for k in range(M32):
            @pl.when(k < m32_out)
            def _(k=k):
                desc32(k).wait_send()
        for j in range(M8):
            @pl.when(j < rem8_out)
            def _(j=j):
                desc8(j).wait_send()

    return pl.pallas_call(
        body,
        out_shape=jax.ShapeDtypeStruct((T, d), out_dtype),
        in_specs=[
            pl.BlockSpec(memory_space=pltpu.SMEM),
            pl.BlockSpec(memory_space=pltpu.VMEM),
            pl.BlockSpec(memory_space=pltpu.VMEM),
            pl.BlockSpec(memory_space=pltpu.VMEM),
        ],
        out_specs=pl.BlockSpec(memory_space=pltpu.VMEM),
        scratch_shapes=[
            pltpu.VMEM((SND_ROWS, d), jnp.bfloat16),
            pltpu.VMEM((SND_ROWS, d), jnp.bfloat16),
            pltpu.SemaphoreType.DMA((M32,)),
            pltpu.SemaphoreType.DMA((M32,)),
            pltpu.SemaphoreType.DMA((M8,)),
            pltpu.SemaphoreType.DMA((M8,)),
        ],
        compiler_params=pltpu.CompilerParams(collective_id=0),
    )(c0_arr, k_idx, s_idx, xb)


def kernel(x, dest):
    t, d = x.shape
    my_y = lax.axis_index("y")
    y0 = my_y == 0

    zeros = (dest == 0).astype(jnp.int32)
    c = jnp.cumsum(zeros)
    c0 = c[-1]
    i = jnp.arange(t, dtype=jnp.int32)
    cum1 = (i + 1) - c
    r_dst = jnp.where(y0, 0, (t - c0) % 8)

    keep_mask = jnp.where(y0, zeros == 1, zeros == 0)
    fwd_k = jnp.where(keep_mask,
                      jnp.where(y0, c - 1, c0 + cum1 - 1), -1)
    fwd_s = jnp.where(keep_mask, -1,
                      jnp.where(y0, cum1 - 1, r_dst + c - 1))

    return _exchange(x, fwd_k.reshape(1, t),
                     fwd_s.reshape(1, t), jnp.reshape(c0, (1,)), x.dtype)
